# baseline (speedup 1.0000x reference)
import functools

import jax
import jax.numpy as jnp
from jax import lax
from jax.experimental import pallas as pl
from jax.experimental.pallas import tpu as pltpu
from jax.experimental.pallas import tpu_sc as plsc

B = 8
N = 4096
S = 1024
K = 32
C_IN = 16
R2 = 0.2 * 0.2
NT = float(B * S * K)


def _fps_kernel(x_ref, y_ref, z_ref, cx_ref, cy_ref, cz_ref):
    x = x_ref[...]
    y = y_ref[...]
    z = z_ref[...]
    lane = lax.broadcasted_iota(jnp.int32, (B, N), 1)
    lane_s = lax.broadcasted_iota(jnp.int32, (B, S), 1)

    def body(i, carry):
        dists, far, cxs, cys, czs = carry
        sel = lane == far
        cx = jnp.sum(jnp.where(sel, x, 0.0), axis=1, keepdims=True)
        cy = jnp.sum(jnp.where(sel, y, 0.0), axis=1, keepdims=True)
        cz = jnp.sum(jnp.where(sel, z, 0.0), axis=1, keepdims=True)
        out_sel = lane_s == i
        cxs = jnp.where(out_sel, cx, cxs)
        cys = jnp.where(out_sel, cy, cys)
        czs = jnp.where(out_sel, cz, czs)
        dx = x - cx
        dy = y - cy
        dz = z - cz
        d = dx * dx + dy * dy + dz * dz
        dists = jnp.minimum(dists, d)
        m = jnp.max(dists, axis=1, keepdims=True)
        far = jnp.min(jnp.where(dists == m, lane, N), axis=1, keepdims=True)
        return dists, far, cxs, cys, czs

    dists0 = jnp.full((B, N), 1e10, jnp.float32)
    far0 = jnp.zeros((B, 1), jnp.int32)
    zero_s = jnp.zeros((B, S), jnp.float32)
    _, _, cxs, cys, czs = lax.fori_loop(
        0, S, body, (dists0, far0, zero_s, zero_s, zero_s))
    cx_ref[...] = cxs
    cy_ref[...] = cys
    cz_ref[...] = czs


def _fps_call(xs, ys, zs):
    out = [jax.ShapeDtypeStruct((B, S), jnp.float32)] * 3
    return pl.pallas_call(
        _fps_kernel,
        out_shape=out,
    )(xs, ys, zs)


ST = 512


def _extract(cand, first, out_ref, base):
    m = first
    for k in range(1, K):
        m = jnp.min(jnp.where(cand > m, cand, N), axis=0, keepdims=True)
        out_ref[0, k:k + 1, :] = jnp.where(m == N, first, m) + base


def _ball_kernel(pxc_ref, pyc_ref, pzc_ref, cx_ref, cy_ref, cz_ref, out_ref):
    b = pl.program_id(0)
    blane = lax.broadcasted_iota(jnp.int32, (N, B), 1) == b
    px = jnp.sum(jnp.where(blane, pxc_ref[...], 0.0), axis=1, keepdims=True)
    py = jnp.sum(jnp.where(blane, pyc_ref[...], 0.0), axis=1, keepdims=True)
    pz = jnp.sum(jnp.where(blane, pzc_ref[...], 0.0), axis=1, keepdims=True)
    cx = cx_ref[0]
    cy = cy_ref[0]
    cz = cz_ref[0]
    dx = px - cx
    dy = py - cy
    dz = pz - cz
    d = dx * dx + dy * dy + dz * dz
    mask = d <= R2
    sub = lax.broadcasted_iota(jnp.int32, (N, ST), 0)
    cand = jnp.where(mask, sub, N)
    first = jnp.minimum(jnp.min(cand, axis=0, keepdims=True), N - 1)
    base = b * N
    out_ref[0, 0:1, :] = first + base
    _extract(cand, first, out_ref, base)


def _ball_call(pxc, pyc, pzc, cxp, cyp, czp):
    grid = (B, S // ST)
    pt_spec = pl.BlockSpec((N, B), lambda b, st: (0, 0))
    c_spec = pl.BlockSpec((1, 1, ST), lambda b, st: (b, 0, st))
    out_spec = pl.BlockSpec((1, K, ST), lambda b, st: (b, 0, st))
    return pl.pallas_call(
        _ball_kernel,
        grid=grid,
        in_specs=[pt_spec, pt_spec, pt_spec, c_spec, c_spec, c_spec],
        out_specs=out_spec,
        out_shape=jax.ShapeDtypeStruct((B, K, S), jnp.int32),
    )(pxc, pyc, pzc, cxp, cyp, czp)


_ROWS = B * K * S
_CHUNK = 128
_D = 128


@functools.cache
def _make_gather_sc():
    info = plsc.get_sparse_core_info()
    nc, ns = info.num_cores, info.num_subcores
    rows_per_w = _ROWS // (nc * ns)

    @functools.partial(
        pl.kernel,
        mesh=plsc.VectorSubcoreMesh(core_axis_name="c", subcore_axis_name="s"),
        out_type=jax.ShapeDtypeStruct((_ROWS, _D), jnp.float32),
        scratch_types=[
            pltpu.VMEM((_CHUNK,), jnp.int32),
            pltpu.VMEM((_CHUNK, _D), jnp.float32),
            pltpu.SemaphoreType.DMA,
        ],
    )
    def gather_sc(table_hbm, idx_hbm, out_hbm, idx_v, rows_v, sem):
        wid = lax.axis_index("s") * nc + lax.axis_index("c")
        base = wid * rows_per_w

        def body(j, carry):
            off = base + j * _CHUNK
            pltpu.sync_copy(idx_hbm.at[pl.dslice(off, _CHUNK)], idx_v)
            pltpu.async_copy(table_hbm.at[idx_v], rows_v, sem).wait()
            pltpu.sync_copy(rows_v, out_hbm.at[pl.dslice(off, _CHUNK)])
            return carry

        lax.fori_loop(0, rows_per_w // _CHUNK, body, 0)

    return gather_sc


def _gather(table, flat_idx):
    return _make_gather_sc()(table, flat_idx)


def _accum_stats(stats_ref, h, is_first):
    s = jnp.sum(h, axis=0, keepdims=True)
    ss = jnp.sum(h * h, axis=0, keepdims=True)

    @pl.when(is_first)
    def _():
        stats_ref[...] = jnp.zeros_like(stats_ref)

    stats_ref[0:1, :] += s
    stats_ref[1:2, :] += ss


KB = 16


def _passA_kernel(g_ref, cmat_ref, w0_ref, stats_ref, xc_ref):
    b = pl.program_id(0)
    k = pl.program_id(1)
    x = g_ref[0][:, :, :32] - cmat_ref[...]
    xc_ref[0] = x
    h0 = jnp.dot(x.reshape(KB * S, 32), w0_ref[...],
                 preferred_element_type=jnp.float32)
    _accum_stats(stats_ref, h0, jnp.logical_and(b == 0, k == 0))


def _chain01(xc_ref, w0_ref, coef0_ref):
    h0 = jnp.dot(xc_ref[0].reshape(KB * S, 32), w0_ref[...],
                 preferred_element_type=jnp.float32)
    return jnp.maximum(h0 * coef0_ref[0:1, :] + coef0_ref[1:2, :], 0.0)


def _passB_kernel(xc_ref, w0_ref, coef0_ref, w1_ref, stats_ref):
    b = pl.program_id(0)
    k = pl.program_id(1)
    g0 = _chain01(xc_ref, w0_ref, coef0_ref)
    h1 = jnp.dot(g0, w1_ref[...], preferred_element_type=jnp.float32)
    _accum_stats(stats_ref, h1, jnp.logical_and(b == 0, k == 0))


def _passC_kernel(xc_ref, w0_ref, coef0_ref, w1_ref, coef1_ref, w2_ref,
                  stats_ref):
    b = pl.program_id(0)
    k = pl.program_id(1)
    g0 = _chain01(xc_ref, w0_ref, coef0_ref)
    h1 = jnp.dot(g0, w1_ref[...], preferred_element_type=jnp.float32)
    g1 = jnp.maximum(h1 * coef1_ref[0:1, :] + coef1_ref[1:2, :], 0.0)
    h2 = jnp.dot(g1, w2_ref[...], preferred_element_type=jnp.float32)
    _accum_stats(stats_ref, h2, jnp.logical_and(b == 0, k == 0))


def _passD_kernel(xc_ref, w0_ref, coef0_ref, w1_ref, coef1_ref, w2_ref,
                  coef2_ref, out_ref):
    k = pl.program_id(1)
    g0 = _chain01(xc_ref, w0_ref, coef0_ref)
    h1 = jnp.dot(g0, w1_ref[...], preferred_element_type=jnp.float32)
    g1 = jnp.maximum(h1 * coef1_ref[0:1, :] + coef1_ref[1:2, :], 0.0)
    h2 = jnp.dot(g1, w2_ref[...], preferred_element_type=jnp.float32)
    g2 = jnp.maximum(h2 * coef2_ref[0:1, :] + coef2_ref[1:2, :], 0.0)

    g2m = jnp.max(g2.reshape(KB, S, 64), axis=0)

    @pl.when(k == 0)
    def _():
        out_ref[0] = g2m

    @pl.when(k != 0)
    def _():
        out_ref[0] = jnp.maximum(out_ref[0], g2m)


def _small_spec(shape):
    return pl.BlockSpec(shape, lambda b, k: tuple(0 for _ in shape))


_G_SPEC = pl.BlockSpec((1, KB, S, _D), lambda b, k: (b, k, 0, 0))
_XC_SPEC = pl.BlockSpec((1, KB, S, 32), lambda b, k: (b, k, 0, 0))
_CMAT_SPEC = pl.BlockSpec((1, S, 32), lambda b, k: (b, 0, 0))


def _run_pass(body, n_out_ch, xc, *smalls):
    specs = [_XC_SPEC] + [_small_spec(s.shape) for s in smalls]
    return pl.pallas_call(
        body,
        grid=(B, K // KB),
        in_specs=specs,
        out_specs=_small_spec((8, n_out_ch)),
        out_shape=jax.ShapeDtypeStruct((8, n_out_ch), jnp.float32),
    )(xc, *smalls)


def _bn_coef(stats, gamma, beta):
    mean = stats[0] / NT
    var = stats[1] / NT - mean * mean
    a = gamma * lax.rsqrt(var + 1e-5)
    c = beta - mean * a
    return jnp.stack([a, c], axis=0)


def kernel(points_xyz, features, conv0_w, conv0_b, bn0_g, bn0_b,
           conv1_w, conv1_b, bn1_g, bn1_b, conv2_w, conv2_b, bn2_g, bn2_b):
    del conv0_b, conv1_b, conv2_b
    xs = points_xyz[:, :, 0]
    ys = points_xyz[:, :, 1]
    zs = points_xyz[:, :, 2]

    cx, cy, cz = _fps_call(xs, ys, zs)

    gidx = _ball_call(
        xs.transpose(1, 0), ys.transpose(1, 0), zs.transpose(1, 0),
        cx.reshape(B, 1, S), cy.reshape(B, 1, S), cz.reshape(B, 1, S))

    table = jnp.concatenate(
        [points_xyz, features,
         jnp.zeros((B, N, _D - 3 - C_IN), jnp.float32)], axis=-1)
    table = table.reshape(B * N, _D)
    grouped = _gather(table, gidx.reshape(_ROWS))
    grouped = grouped.reshape(B, K, S, _D)

    cmat = jnp.concatenate(
        [cx[..., None], cy[..., None], cz[..., None],
         jnp.zeros((B, S, 32 - 3), jnp.float32)], axis=-1)

    w0 = jnp.zeros((32, 32), jnp.float32).at[:3 + C_IN].set(conv0_w.T)
    w1 = conv1_w.T
    w2 = conv2_w.T

    stats0, xc = pl.pallas_call(
        _passA_kernel,
        grid=(B, K // KB),
        in_specs=[_G_SPEC, _CMAT_SPEC, _small_spec((32, 32))],
        out_specs=[_small_spec((8, 32)), _XC_SPEC],
        out_shape=[jax.ShapeDtypeStruct((8, 32), jnp.float32),
                   jax.ShapeDtypeStruct((B, K, S, 32), jnp.float32)],
    )(grouped, cmat, w0)
    coef0 = _bn_coef(stats0, bn0_g, bn0_b)
    stats1 = _run_pass(_passB_kernel, 32, xc, w0, coef0, w1)
    coef1 = _bn_coef(stats1, bn1_g, bn1_b)
    stats2 = _run_pass(_passC_kernel, 64, xc, w0, coef0, w1, coef1, w2)
    coef2 = _bn_coef(stats2, bn2_g, bn2_b)

    out_spec = pl.BlockSpec((1, S, 64), lambda b, k: (b, 0, 0))
    g = pl.pallas_call(
        _passD_kernel,
        grid=(B, K // KB),
        in_specs=[_XC_SPEC, _small_spec((32, 32)), _small_spec((2, 32)),
                  _small_spec((32, 32)), _small_spec((2, 32)),
                  _small_spec((32, 64)), _small_spec((2, 64))],
        out_specs=out_spec,
        out_shape=jax.ShapeDtypeStruct((B, S, 64), jnp.float32),
    )(xc, w0, coef0, w1, coef1, w2, coef2)

    centroids = jnp.stack([cx, cy, cz], axis=-1)
    return centroids, g


# speedup vs baseline: 1.1147x; 1.1147x over previous
import functools

import jax
import jax.numpy as jnp
from jax import lax
from jax.experimental import pallas as pl
from jax.experimental.pallas import tpu as pltpu
from jax.experimental.pallas import tpu_sc as plsc

B = 8
N = 4096
S = 1024
K = 32
C_IN = 16
R2 = 0.2 * 0.2
NT = float(B * S * K)


def _fps_kernel(x_ref, y_ref, z_ref, cx_ref, cy_ref, cz_ref):
    x = x_ref[...]
    y = y_ref[...]
    z = z_ref[...]
    lane = lax.broadcasted_iota(jnp.int32, (B, N), 1)
    lane_s = lax.broadcasted_iota(jnp.int32, (B, S), 1)

    def body(i, carry):
        dists, far, cxs, cys, czs = carry
        sel = lane == far
        cx = jnp.sum(jnp.where(sel, x, 0.0), axis=1, keepdims=True)
        cy = jnp.sum(jnp.where(sel, y, 0.0), axis=1, keepdims=True)
        cz = jnp.sum(jnp.where(sel, z, 0.0), axis=1, keepdims=True)
        out_sel = lane_s == i
        cxs = jnp.where(out_sel, cx, cxs)
        cys = jnp.where(out_sel, cy, cys)
        czs = jnp.where(out_sel, cz, czs)
        dx = x - cx
        dy = y - cy
        dz = z - cz
        d = dx * dx + dy * dy + dz * dz
        dists = jnp.minimum(dists, d)
        m = jnp.max(dists, axis=1, keepdims=True)
        far = jnp.min(jnp.where(dists == m, lane, N), axis=1, keepdims=True)
        return dists, far, cxs, cys, czs

    dists0 = jnp.full((B, N), 1e10, jnp.float32)
    far0 = jnp.zeros((B, 1), jnp.int32)
    zero_s = jnp.zeros((B, S), jnp.float32)
    _, _, cxs, cys, czs = lax.fori_loop(
        0, S, body, (dists0, far0, zero_s, zero_s, zero_s))
    cx_ref[...] = cxs
    cy_ref[...] = cys
    cz_ref[...] = czs


def _fps_call(xs, ys, zs):
    out = [jax.ShapeDtypeStruct((B, S), jnp.float32)] * 3
    return pl.pallas_call(
        _fps_kernel,
        out_shape=out,
    )(xs, ys, zs)


ST = 256


def _extract(cand, first, out_ref, base):
    m = first
    for k in range(1, K):
        m = jnp.min(jnp.where(cand > m, cand, N), axis=0, keepdims=True)
        out_ref[0, k:k + 1, :] = jnp.where(m == N, first, m) + base


def _ball_kernel(pxc_ref, pyc_ref, pzc_ref, cx_ref, cy_ref, cz_ref, out_ref):
    b = pl.program_id(0)
    blane = lax.broadcasted_iota(jnp.int32, (N, B), 1) == b
    px = jnp.sum(jnp.where(blane, pxc_ref[...], 0.0), axis=1, keepdims=True)
    py = jnp.sum(jnp.where(blane, pyc_ref[...], 0.0), axis=1, keepdims=True)
    pz = jnp.sum(jnp.where(blane, pzc_ref[...], 0.0), axis=1, keepdims=True)
    cx = cx_ref[0]
    cy = cy_ref[0]
    cz = cz_ref[0]
    dx = px - cx
    dy = py - cy
    dz = pz - cz
    d = dx * dx + dy * dy + dz * dz
    mask = d <= R2
    sub = lax.broadcasted_iota(jnp.int32, (N, ST), 0)
    cand = jnp.where(mask, sub, N)
    first = jnp.minimum(jnp.min(cand, axis=0, keepdims=True), N - 1)
    base = b * N
    out_ref[0, 0:1, :] = first + base
    _extract(cand, first, out_ref, base)


def _ball_call(pxc, pyc, pzc, cxp, cyp, czp):
    grid = (B, S // ST)
    pt_spec = pl.BlockSpec((N, B), lambda b, st: (0, 0))
    c_spec = pl.BlockSpec((1, 1, ST), lambda b, st: (b, 0, st))
    out_spec = pl.BlockSpec((1, K, ST), lambda b, st: (b, 0, st))
    return pl.pallas_call(
        _ball_kernel,
        grid=grid,
        in_specs=[pt_spec, pt_spec, pt_spec, c_spec, c_spec, c_spec],
        out_specs=out_spec,
        out_shape=jax.ShapeDtypeStruct((B, K, S), jnp.int32),
    )(pxc, pyc, pzc, cxp, cyp, czp)


_ROWS = B * K * S
_CHUNK = 128
_D = 128


@functools.cache
def _make_gather_sc():
    info = plsc.get_sparse_core_info()
    nc, ns = info.num_cores, info.num_subcores
    rows_per_w = _ROWS // (nc * ns)

    @functools.partial(
        pl.kernel,
        mesh=plsc.VectorSubcoreMesh(core_axis_name="c", subcore_axis_name="s"),
        out_type=jax.ShapeDtypeStruct((_ROWS, _D), jnp.float32),
        scratch_types=[
            pltpu.VMEM((_CHUNK,), jnp.int32),
            pltpu.VMEM((_CHUNK, _D), jnp.float32),
            pltpu.SemaphoreType.DMA,
        ],
    )
    def gather_sc(table_hbm, idx_hbm, out_hbm, idx_v, rows_v, sem):
        wid = lax.axis_index("s") * nc + lax.axis_index("c")
        base = wid * rows_per_w

        def body(j, carry):
            off = base + j * _CHUNK
            pltpu.sync_copy(idx_hbm.at[pl.dslice(off, _CHUNK)], idx_v)
            pltpu.async_copy(table_hbm.at[idx_v], rows_v, sem).wait()
            pltpu.sync_copy(rows_v, out_hbm.at[pl.dslice(off, _CHUNK)])
            return carry

        lax.fori_loop(0, rows_per_w // _CHUNK, body, 0)

    return gather_sc


def _gather(table, flat_idx):
    return _make_gather_sc()(table, flat_idx)


def _accum_stats(stats_ref, h, is_first):
    s = jnp.sum(h, axis=0, keepdims=True)
    ss = jnp.sum(h * h, axis=0, keepdims=True)

    @pl.when(is_first)
    def _():
        stats_ref[...] = jnp.zeros_like(stats_ref)

    stats_ref[0:1, :] += s
    stats_ref[1:2, :] += ss


KB = 8


def _passA_kernel(g_ref, cmat_ref, w0_ref, stats_ref, xc_ref):
    b = pl.program_id(0)
    k = pl.program_id(1)
    x = g_ref[0][:, :, :32] - cmat_ref[...]
    xc_ref[0] = x
    h0 = jnp.dot(x.reshape(KB * S, 32), w0_ref[...],
                 preferred_element_type=jnp.float32)
    _accum_stats(stats_ref, h0, jnp.logical_and(b == 0, k == 0))


def _chain01(xc_ref, w0_ref, coef0_ref):
    h0 = jnp.dot(xc_ref[0].reshape(KB * S, 32), w0_ref[...],
                 preferred_element_type=jnp.float32)
    return jnp.maximum(h0 * coef0_ref[0:1, :] + coef0_ref[1:2, :], 0.0)


def _passB_kernel(xc_ref, w0_ref, coef0_ref, w1_ref, stats_ref):
    b = pl.program_id(0)
    k = pl.program_id(1)
    g0 = _chain01(xc_ref, w0_ref, coef0_ref)
    h1 = jnp.dot(g0, w1_ref[...], preferred_element_type=jnp.float32)
    _accum_stats(stats_ref, h1, jnp.logical_and(b == 0, k == 0))


def _passC_kernel(xc_ref, w0_ref, coef0_ref, w1_ref, coef1_ref, w2_ref,
                  stats_ref):
    b = pl.program_id(0)
    k = pl.program_id(1)
    g0 = _chain01(xc_ref, w0_ref, coef0_ref)
    h1 = jnp.dot(g0, w1_ref[...], preferred_element_type=jnp.float32)
    g1 = jnp.maximum(h1 * coef1_ref[0:1, :] + coef1_ref[1:2, :], 0.0)
    h2 = jnp.dot(g1, w2_ref[...], preferred_element_type=jnp.float32)
    _accum_stats(stats_ref, h2, jnp.logical_and(b == 0, k == 0))


def _passD_kernel(xc_ref, w0_ref, coef0_ref, w1_ref, coef1_ref, w2_ref,
                  coef2_ref, out_ref):
    k = pl.program_id(1)
    g0 = _chain01(xc_ref, w0_ref, coef0_ref)
    h1 = jnp.dot(g0, w1_ref[...], preferred_element_type=jnp.float32)
    g1 = jnp.maximum(h1 * coef1_ref[0:1, :] + coef1_ref[1:2, :], 0.0)
    h2 = jnp.dot(g1, w2_ref[...], preferred_element_type=jnp.float32)
    g2 = jnp.maximum(h2 * coef2_ref[0:1, :] + coef2_ref[1:2, :], 0.0)

    g2m = jnp.max(g2.reshape(KB, S, 64), axis=0)

    @pl.when(k == 0)
    def _():
        out_ref[0] = g2m

    @pl.when(k != 0)
    def _():
        out_ref[0] = jnp.maximum(out_ref[0], g2m)


def _small_spec(shape):
    return pl.BlockSpec(shape, lambda b, k: tuple(0 for _ in shape))


_G_SPEC = pl.BlockSpec((1, KB, S, _D), lambda b, k: (b, k, 0, 0))
_XC_SPEC = pl.BlockSpec((1, KB, S, 32), lambda b, k: (b, k, 0, 0))
_CMAT_SPEC = pl.BlockSpec((1, S, 32), lambda b, k: (b, 0, 0))


def _run_pass(body, n_out_ch, xc, *smalls):
    specs = [_XC_SPEC] + [_small_spec(s.shape) for s in smalls]
    return pl.pallas_call(
        body,
        grid=(B, K // KB),
        in_specs=specs,
        out_specs=_small_spec((8, n_out_ch)),
        out_shape=jax.ShapeDtypeStruct((8, n_out_ch), jnp.float32),
    )(xc, *smalls)


def _bn_coef(stats, gamma, beta):
    mean = stats[0] / NT
    var = stats[1] / NT - mean * mean
    a = gamma * lax.rsqrt(var + 1e-5)
    c = beta - mean * a
    return jnp.stack([a, c], axis=0)


def kernel(points_xyz, features, conv0_w, conv0_b, bn0_g, bn0_b,
           conv1_w, conv1_b, bn1_g, bn1_b, conv2_w, conv2_b, bn2_g, bn2_b):
    del conv0_b, conv1_b, conv2_b
    xs = points_xyz[:, :, 0]
    ys = points_xyz[:, :, 1]
    zs = points_xyz[:, :, 2]

    cx, cy, cz = _fps_call(xs, ys, zs)

    gidx = _ball_call(
        xs.transpose(1, 0), ys.transpose(1, 0), zs.transpose(1, 0),
        cx.reshape(B, 1, S), cy.reshape(B, 1, S), cz.reshape(B, 1, S))

    table = jnp.concatenate(
        [points_xyz, features,
         jnp.zeros((B, N, _D - 3 - C_IN), jnp.float32)], axis=-1)
    table = table.reshape(B * N, _D)
    grouped = _gather(table, gidx.reshape(_ROWS))
    grouped = grouped.reshape(B, K, S, _D)

    cmat = jnp.concatenate(
        [cx[..., None], cy[..., None], cz[..., None],
         jnp.zeros((B, S, 32 - 3), jnp.float32)], axis=-1)

    w0 = jnp.zeros((32, 32), jnp.float32).at[:3 + C_IN].set(conv0_w.T)
    w1 = conv1_w.T
    w2 = conv2_w.T

    stats0, xc = pl.pallas_call(
        _passA_kernel,
        grid=(B, K // KB),
        in_specs=[_G_SPEC, _CMAT_SPEC, _small_spec((32, 32))],
        out_specs=[_small_spec((8, 32)), _XC_SPEC],
        out_shape=[jax.ShapeDtypeStruct((8, 32), jnp.float32),
                   jax.ShapeDtypeStruct((B, K, S, 32), jnp.float32)],
    )(grouped, cmat, w0)
    coef0 = _bn_coef(stats0, bn0_g, bn0_b)
    stats1 = _run_pass(_passB_kernel, 32, xc, w0, coef0, w1)
    coef1 = _bn_coef(stats1, bn1_g, bn1_b)
    stats2 = _run_pass(_passC_kernel, 64, xc, w0, coef0, w1, coef1, w2)
    coef2 = _bn_coef(stats2, bn2_g, bn2_b)

    out_spec = pl.BlockSpec((1, S, 64), lambda b, k: (b, 0, 0))
    g = pl.pallas_call(
        _passD_kernel,
        grid=(B, K // KB),
        in_specs=[_XC_SPEC, _small_spec((32, 32)), _small_spec((2, 32)),
                  _small_spec((32, 32)), _small_spec((2, 32)),
                  _small_spec((32, 64)), _small_spec((2, 64))],
        out_specs=out_spec,
        out_shape=jax.ShapeDtypeStruct((B, S, 64), jnp.float32),
    )(xc, w0, coef0, w1, coef1, w2, coef2)

    centroids = jnp.stack([cx, cy, cz], axis=-1)
    return centroids, g


# speedup vs baseline: 1.1452x; 1.0274x over previous
import functools

import jax
import jax.numpy as jnp
from jax import lax
from jax.experimental import pallas as pl
from jax.experimental.pallas import tpu as pltpu
from jax.experimental.pallas import tpu_sc as plsc

B = 8
N = 4096
S = 1024
K = 32
C_IN = 16
R2 = 0.2 * 0.2
NT = float(B * S * K)


def _fps_kernel(x_ref, y_ref, z_ref, cx_ref, cy_ref, cz_ref):
    x = x_ref[...]
    y = y_ref[...]
    z = z_ref[...]
    lane = lax.broadcasted_iota(jnp.int32, (B, N), 1)
    lane_s = lax.broadcasted_iota(jnp.int32, (B, S), 1)

    def body(i, carry):
        dists, far, cxs, cys, czs = carry
        sel = lane == far
        cx = jnp.sum(jnp.where(sel, x, 0.0), axis=1, keepdims=True)
        cy = jnp.sum(jnp.where(sel, y, 0.0), axis=1, keepdims=True)
        cz = jnp.sum(jnp.where(sel, z, 0.0), axis=1, keepdims=True)
        out_sel = lane_s == i
        cxs = jnp.where(out_sel, cx, cxs)
        cys = jnp.where(out_sel, cy, cys)
        czs = jnp.where(out_sel, cz, czs)
        dx = x - cx
        dy = y - cy
        dz = z - cz
        d = dx * dx + dy * dy + dz * dz
        dists = jnp.minimum(dists, d)
        m = jnp.max(dists, axis=1, keepdims=True)
        far = jnp.min(jnp.where(dists == m, lane, N), axis=1, keepdims=True)
        return dists, far, cxs, cys, czs

    dists0 = jnp.full((B, N), 1e10, jnp.float32)
    far0 = jnp.zeros((B, 1), jnp.int32)
    zero_s = jnp.zeros((B, S), jnp.float32)
    _, _, cxs, cys, czs = lax.fori_loop(
        0, S, body, (dists0, far0, zero_s, zero_s, zero_s))
    cx_ref[...] = cxs
    cy_ref[...] = cys
    cz_ref[...] = czs


def _fps_call(xs, ys, zs):
    out = [jax.ShapeDtypeStruct((B, S), jnp.float32)] * 3
    return pl.pallas_call(
        _fps_kernel,
        out_shape=out,
    )(xs, ys, zs)


ST = 256


def _extract(cand, first, out_ref, base):
    m = first
    for k in range(1, K):
        m = jnp.min(jnp.where(cand > m, cand, N), axis=0, keepdims=True)
        out_ref[0, k:k + 1, :] = jnp.where(m == N, first, m) + base


def _ball_kernel(pxc_ref, pyc_ref, pzc_ref, cx_ref, cy_ref, cz_ref, out_ref):
    b = pl.program_id(0)
    blane = lax.broadcasted_iota(jnp.int32, (N, B), 1) == b
    px = jnp.sum(jnp.where(blane, pxc_ref[...], 0.0), axis=1, keepdims=True)
    py = jnp.sum(jnp.where(blane, pyc_ref[...], 0.0), axis=1, keepdims=True)
    pz = jnp.sum(jnp.where(blane, pzc_ref[...], 0.0), axis=1, keepdims=True)
    cx = cx_ref[0]
    cy = cy_ref[0]
    cz = cz_ref[0]
    dx = px - cx
    dy = py - cy
    dz = pz - cz
    d = dx * dx + dy * dy + dz * dz
    mask = d <= R2
    sub = lax.broadcasted_iota(jnp.int32, (N, ST), 0)
    cand = jnp.where(mask, sub, N)
    first = jnp.minimum(jnp.min(cand, axis=0, keepdims=True), N - 1)
    base = b * N
    out_ref[0, 0:1, :] = first + base
    _extract(cand, first, out_ref, base)


def _ball_call(pxc, pyc, pzc, cxp, cyp, czp):
    grid = (B, S // ST)
    pt_spec = pl.BlockSpec((N, B), lambda b, st: (0, 0))
    c_spec = pl.BlockSpec((1, 1, ST), lambda b, st: (b, 0, st))
    out_spec = pl.BlockSpec((1, K, ST), lambda b, st: (b, 0, st))
    return pl.pallas_call(
        _ball_kernel,
        grid=grid,
        in_specs=[pt_spec, pt_spec, pt_spec, c_spec, c_spec, c_spec],
        out_specs=out_spec,
        out_shape=jax.ShapeDtypeStruct((B, K, S), jnp.int32),
    )(pxc, pyc, pzc, cxp, cyp, czp)


_ROWS = B * K * S
_CHUNK = 128
_D = 128


@functools.cache
def _make_gather_sc():
    info = plsc.get_sparse_core_info()
    nc, ns = info.num_cores, info.num_subcores
    rows_per_w = _ROWS // (nc * ns)

    @functools.partial(
        pl.kernel,
        mesh=plsc.VectorSubcoreMesh(core_axis_name="c", subcore_axis_name="s"),
        out_type=jax.ShapeDtypeStruct((_ROWS, _D), jnp.float32),
        scratch_types=[
            pltpu.VMEM((2 * _CHUNK,), jnp.int32),
            pltpu.VMEM((2 * _CHUNK, _D), jnp.float32),
            pltpu.SemaphoreType.DMA,
            pltpu.SemaphoreType.DMA,
        ],
    )
    def gather_sc(table_hbm, idx_hbm, out_hbm, idx_v, rows_v, sem_a, sem_b):
        wid = lax.axis_index("s") * nc + lax.axis_index("c")
        base = wid * rows_per_w

        def body(j, carry):
            off = base + j * 2 * _CHUNK
            pltpu.sync_copy(idx_hbm.at[pl.dslice(off, 2 * _CHUNK)], idx_v)
            ca = pltpu.async_copy(
                table_hbm.at[idx_v.at[pl.dslice(0, _CHUNK)]],
                rows_v.at[pl.dslice(0, _CHUNK)], sem_a)
            cb = pltpu.async_copy(
                table_hbm.at[idx_v.at[pl.dslice(_CHUNK, _CHUNK)]],
                rows_v.at[pl.dslice(_CHUNK, _CHUNK)], sem_b)
            ca.wait()
            cb.wait()
            pltpu.sync_copy(rows_v, out_hbm.at[pl.dslice(off, 2 * _CHUNK)])
            return carry

        lax.fori_loop(0, rows_per_w // (2 * _CHUNK), body, 0)

    return gather_sc


def _gather(table, flat_idx):
    return _make_gather_sc()(table, flat_idx)


def _accum_stats(stats_ref, h, is_first):
    s = jnp.sum(h, axis=0, keepdims=True)
    ss = jnp.sum(h * h, axis=0, keepdims=True)

    @pl.when(is_first)
    def _():
        stats_ref[...] = jnp.zeros_like(stats_ref)

    stats_ref[0:1, :] += s
    stats_ref[1:2, :] += ss


KB = 8


def _passA_kernel(g_ref, cmat_ref, w0_ref, stats_ref, xc_ref):
    b = pl.program_id(0)
    k = pl.program_id(1)
    x = g_ref[0][:, :, :32] - cmat_ref[...]
    xc_ref[0] = x
    h0 = jnp.dot(x.reshape(KB * S, 32), w0_ref[...],
                 preferred_element_type=jnp.float32)
    _accum_stats(stats_ref, h0, jnp.logical_and(b == 0, k == 0))


def _chain01(xc_ref, w0_ref, coef0_ref):
    h0 = jnp.dot(xc_ref[0].reshape(KB * S, 32), w0_ref[...],
                 preferred_element_type=jnp.float32)
    return jnp.maximum(h0 * coef0_ref[0:1, :] + coef0_ref[1:2, :], 0.0)


def _passB_kernel(xc_ref, w0_ref, coef0_ref, w1_ref, stats_ref):
    b = pl.program_id(0)
    k = pl.program_id(1)
    g0 = _chain01(xc_ref, w0_ref, coef0_ref)
    h1 = jnp.dot(g0, w1_ref[...], preferred_element_type=jnp.float32)
    _accum_stats(stats_ref, h1, jnp.logical_and(b == 0, k == 0))


def _passC_kernel(xc_ref, w0_ref, coef0_ref, w1_ref, coef1_ref, w2_ref,
                  stats_ref):
    b = pl.program_id(0)
    k = pl.program_id(1)
    g0 = _chain01(xc_ref, w0_ref, coef0_ref)
    h1 = jnp.dot(g0, w1_ref[...], preferred_element_type=jnp.float32)
    g1 = jnp.maximum(h1 * coef1_ref[0:1, :] + coef1_ref[1:2, :], 0.0)
    h2 = jnp.dot(g1, w2_ref[...], preferred_element_type=jnp.float32)
    _accum_stats(stats_ref, h2, jnp.logical_and(b == 0, k == 0))


def _passD_kernel(xc_ref, w0_ref, coef0_ref, w1_ref, coef1_ref, w2_ref,
                  coef2_ref, out_ref):
    k = pl.program_id(1)
    g0 = _chain01(xc_ref, w0_ref, coef0_ref)
    h1 = jnp.dot(g0, w1_ref[...], preferred_element_type=jnp.float32)
    g1 = jnp.maximum(h1 * coef1_ref[0:1, :] + coef1_ref[1:2, :], 0.0)
    h2 = jnp.dot(g1, w2_ref[...], preferred_element_type=jnp.float32)
    g2 = jnp.maximum(h2 * coef2_ref[0:1, :] + coef2_ref[1:2, :], 0.0)

    g2m = jnp.max(g2.reshape(KB, S, 64), axis=0)

    @pl.when(k == 0)
    def _():
        out_ref[0] = g2m

    @pl.when(k != 0)
    def _():
        out_ref[0] = jnp.maximum(out_ref[0], g2m)


def _small_spec(shape):
    return pl.BlockSpec(shape, lambda b, k: tuple(0 for _ in shape))


_G_SPEC = pl.BlockSpec((1, KB, S, _D), lambda b, k: (b, k, 0, 0))
_XC_SPEC = pl.BlockSpec((1, KB, S, 32), lambda b, k: (b, k, 0, 0))
_CMAT_SPEC = pl.BlockSpec((1, S, 32), lambda b, k: (b, 0, 0))


def _run_pass(body, n_out_ch, xc, *smalls):
    specs = [_XC_SPEC] + [_small_spec(s.shape) for s in smalls]
    return pl.pallas_call(
        body,
        grid=(B, K // KB),
        in_specs=specs,
        out_specs=_small_spec((8, n_out_ch)),
        out_shape=jax.ShapeDtypeStruct((8, n_out_ch), jnp.float32),
    )(xc, *smalls)


def _bn_coef(stats, gamma, beta):
    mean = stats[0] / NT
    var = stats[1] / NT - mean * mean
    a = gamma * lax.rsqrt(var + 1e-5)
    c = beta - mean * a
    return jnp.stack([a, c], axis=0)


def kernel(points_xyz, features, conv0_w, conv0_b, bn0_g, bn0_b,
           conv1_w, conv1_b, bn1_g, bn1_b, conv2_w, conv2_b, bn2_g, bn2_b):
    del conv0_b, conv1_b, conv2_b
    xs = points_xyz[:, :, 0]
    ys = points_xyz[:, :, 1]
    zs = points_xyz[:, :, 2]

    cx, cy, cz = _fps_call(xs, ys, zs)

    gidx = _ball_call(
        xs.transpose(1, 0), ys.transpose(1, 0), zs.transpose(1, 0),
        cx.reshape(B, 1, S), cy.reshape(B, 1, S), cz.reshape(B, 1, S))

    table = jnp.concatenate(
        [points_xyz, features,
         jnp.zeros((B, N, _D - 3 - C_IN), jnp.float32)], axis=-1)
    table = table.reshape(B * N, _D)
    grouped = _gather(table, gidx.reshape(_ROWS))
    grouped = grouped.reshape(B, K, S, _D)

    cmat = jnp.concatenate(
        [cx[..., None], cy[..., None], cz[..., None],
         jnp.zeros((B, S, 32 - 3), jnp.float32)], axis=-1)

    w0 = jnp.zeros((32, 32), jnp.float32).at[:3 + C_IN].set(conv0_w.T)
    w1 = conv1_w.T
    w2 = conv2_w.T

    stats0, xc = pl.pallas_call(
        _passA_kernel,
        grid=(B, K // KB),
        in_specs=[_G_SPEC, _CMAT_SPEC, _small_spec((32, 32))],
        out_specs=[_small_spec((8, 32)), _XC_SPEC],
        out_shape=[jax.ShapeDtypeStruct((8, 32), jnp.float32),
                   jax.ShapeDtypeStruct((B, K, S, 32), jnp.float32)],
    )(grouped, cmat, w0)
    coef0 = _bn_coef(stats0, bn0_g, bn0_b)
    stats1 = _run_pass(_passB_kernel, 32, xc, w0, coef0, w1)
    coef1 = _bn_coef(stats1, bn1_g, bn1_b)
    stats2 = _run_pass(_passC_kernel, 64, xc, w0, coef0, w1, coef1, w2)
    coef2 = _bn_coef(stats2, bn2_g, bn2_b)

    out_spec = pl.BlockSpec((1, S, 64), lambda b, k: (b, 0, 0))
    g = pl.pallas_call(
        _passD_kernel,
        grid=(B, K // KB),
        in_specs=[_XC_SPEC, _small_spec((32, 32)), _small_spec((2, 32)),
                  _small_spec((32, 32)), _small_spec((2, 32)),
                  _small_spec((32, 64)), _small_spec((2, 64))],
        out_specs=out_spec,
        out_shape=jax.ShapeDtypeStruct((B, S, 64), jnp.float32),
    )(xc, w0, coef0, w1, coef1, w2, coef2)

    centroids = jnp.stack([cx, cy, cz], axis=-1)
    return centroids, g
